# baseline (speedup 1.0000x reference)
import functools

import jax
import jax.numpy as jnp
from jax import lax
from jax.experimental import pallas as pl
from jax.experimental.pallas import tpu as pltpu
from jax.experimental.pallas import tpu_sc as plsc

B = 2
G = 128
C = 8
NP = 262144
TOTAL = B * NP
NW = 32
PER_W = TOTAL // NW
CHUNK = 512
NCHUNK = PER_W // CHUNK
NGRP = CHUNK // 16
GSLICE = 128
NSLICE = CHUNK * 8 // GSLICE
ROWS = B * G * G * G

_CORNERS = [(dx, dy, dz) for dx in (0, 1) for dy in (0, 1) for dz in (0, 1)]


def _axis_split(v):
    f = v * 128.0 - 0.5
    t = f.astype(jnp.int32)
    tf = t.astype(jnp.float32)
    neg = f < tf
    i0 = jnp.where(neg, t - 1, t)
    i0f = jnp.where(neg, tf - 1.0, tf)
    frac = f - i0f
    c0 = jnp.maximum(i0, 0)
    c1 = jnp.minimum(i0 + 1, G - 1)
    return c0, c1, frac


@functools.partial(
    pl.kernel,
    mesh=plsc.VectorSubcoreMesh(core_axis_name="c", subcore_axis_name="s"),
    out_type=jax.ShapeDtypeStruct((TOTAL * C,), jnp.float32),
    compiler_params=pltpu.CompilerParams(
        needs_layout_passes=False, use_tc_tiling_on_sc=False
    ),
    scratch_types=[
        pltpu.VMEM((CHUNK,), jnp.float32),
        pltpu.VMEM((CHUNK,), jnp.float32),
        pltpu.VMEM((CHUNK,), jnp.float32),
        pltpu.VMEM((8 * CHUNK,), jnp.int32),
        pltpu.VMEM((8 * CHUNK,), jnp.float32),
        pltpu.VMEM((8 * CHUNK, C), jnp.float32),
        pltpu.VMEM((CHUNK * C,), jnp.float32),
        pltpu.SemaphoreType.DMA,
    ],
)
def _sc_interp(rows_ref, xs_ref, ys_ref, zs_ref, out_ref,
               xs_v, ys_v, zs_v, idx_v, w_v, rows_v, out_v, sem):
    cid = lax.axis_index("c")
    sid = lax.axis_index("s")
    wid = sid * 2 + cid
    wbase = wid * PER_W
    boff = (wbase // NP) * (G * G * G)
    iota = lax.iota(jnp.int32, 16)
    iota8 = iota * C
    cols = [jnp.full((16,), c, jnp.int32) for c in range(C)]

    def chunk_body(n, carry):
        base = wbase + n * CHUNK
        pltpu.sync_copy(xs_ref.at[pl.ds(base, CHUNK)], xs_v)
        pltpu.sync_copy(ys_ref.at[pl.ds(base, CHUNK)], ys_v)
        pltpu.sync_copy(zs_ref.at[pl.ds(base, CHUNK)], zs_v)

        def grp_idx(j, c2):
            o = j * 16
            x0, x1, tx = _axis_split(xs_v[pl.ds(o, 16)])
            y0, y1, ty = _axis_split(ys_v[pl.ds(o, 16)])
            z0, z1, tz = _axis_split(zs_v[pl.ds(o, 16)])
            cx = (x0 * (G * G), x1 * (G * G))
            cy = (y0 * G, y1 * G)
            cz = (z0 + boff, z1 + boff)
            ux = (1.0 - tx, tx)
            uy = (1.0 - ty, ty)
            uz = (1.0 - tz, tz)
            wxy = {(a, b): ux[a] * uy[b] for a in (0, 1) for b in (0, 1)}
            for k, (dx, dy, dz) in enumerate(_CORNERS):
                idx_v[pl.ds(k * CHUNK + o, 16)] = cx[dx] + cy[dy] + cz[dz]
                w_v[pl.ds(k * CHUNK + o, 16)] = wxy[(dx, dy)] * uz[dz]
            return c2

        lax.fori_loop(0, NGRP, grp_idx, 0)

        cps = [
            pltpu.async_copy(
                rows_ref.at[idx_v.at[pl.ds(s * GSLICE, GSLICE)]],
                rows_v.at[pl.ds(s * GSLICE, GSLICE)],
                sem,
            )
            for s in range(NSLICE)
        ]
        for cp in cps:
            cp.wait()

        def grp_acc(j, c2):
            o = j * 16
            accs = [jnp.zeros((16,), jnp.float32) for _ in range(C)]
            for k in range(8):
                w = w_v[pl.ds(k * CHUNK + o, 16)]
                ri = iota + (k * CHUNK + o)
                for c in range(C):
                    vals = plsc.load_gather(rows_v, [ri, cols[c]])
                    accs[c] = accs[c] + w * vals
            ro = iota8 + o * C
            for c in range(C):
                plsc.store_scatter(out_v, [ro + c], accs[c])
            return c2

        lax.fori_loop(0, NGRP, grp_acc, 0)
        pltpu.sync_copy(out_v, out_ref.at[pl.ds(base * C, CHUNK * C)])
        return carry

    lax.fori_loop(0, NCHUNK, chunk_body, 0)


def kernel(grid, locs):
    rows = grid.reshape(ROWS, C)
    lf = locs.reshape(TOTAL, 4)
    out = _sc_interp(rows, lf[:, 0], lf[:, 1], lf[:, 2])
    return out.reshape(B, NP, C)
